# Initial kernel scaffold; baseline (speedup 1.0000x reference)
import jax
import jax.numpy as jnp
from jax.experimental import pallas as pl


def kernel(x, adj, W0, a0, W1, a1, W2, a2, W3, a3, W_out, a_out):
    raise NotImplementedError("write your pallas kernel here")


# speedup vs baseline: 5.1883x; 5.1883x over previous
import functools

import jax
import jax.numpy as jnp
from jax import lax
from jax.experimental import pallas as pl
from jax.experimental.pallas import tpu as pltpu
from jax.experimental.pallas import tpu_sc as plsc

N_NODES = 10000
N_PAD = 10240
E_EDGES = 320000
NFEAT = 128
NHID = 64
NHEADS = 4
NCLASS = 16
ALPHA = 0.2

NC = 2
NS = 16
LANES = 16

HPW = 2 * NHID
W2ROW = 2 * NCLASS

TCB = 1024
NBLK = N_PAD // TCB


def _tc1_body(x_ref, wc_ref, alm_ref, erm_ref, h_ref, ltab_ref, rtab_ref):
    h = jnp.dot(x_ref[...], wc_ref[0], preferred_element_type=jnp.float32)
    h_ref[...] = h
    ltab_ref[...] = jnp.dot(h, alm_ref[0], preferred_element_type=jnp.float32)
    rtab_ref[...] = jnp.dot(h, erm_ref[0], preferred_element_type=jnp.float32)


def _tc1(xp, Wc, ALm, ERm):
    return pl.pallas_call(
        _tc1_body,
        grid=(NC, NBLK),
        in_specs=[
            pl.BlockSpec((TCB, NFEAT), lambda c, i: (i, 0)),
            pl.BlockSpec((1, NFEAT, HPW), lambda c, i: (c, 0, 0)),
            pl.BlockSpec((1, HPW, 16), lambda c, i: (c, 0, 0)),
            pl.BlockSpec((1, HPW, 16), lambda c, i: (c, 0, 0)),
        ],
        out_specs=[
            pl.BlockSpec((TCB, HPW), lambda c, i: (c * NBLK + i, 0)),
            pl.BlockSpec((TCB, 16), lambda c, i: (c * NBLK + i, 0)),
            pl.BlockSpec((TCB, 16), lambda c, i: (c * NBLK + i, 0)),
        ],
        out_shape=[
            jax.ShapeDtypeStruct((NC * N_PAD, HPW), jnp.float32),
            jax.ShapeDtypeStruct((NC * N_PAD, 16), jnp.float32),
            jax.ShapeDtypeStruct((NC * N_PAD, 16), jnp.float32),
        ],
    )(xp, Wc, ALm, ERm)


EPT1 = E_EDGES // NS
C1 = 80
NCH1 = EPT1 // C1
GRP1 = C1 // LANES
RPT = N_PAD // NS


def _sc1_body(htab, ltab, rtab, srcs, dsts, zd, zw, outd, outw,
              sbuf, dbuf, sobuf, dobuf, abuf, bbuf, rbuf, obuf, wtbuf,
              accd, accw, sem1, sem2):
    cid = lax.axis_index("c")
    sid = lax.axis_index("s")
    pltpu.sync_copy(zd, accd.at[pl.ds(sid * RPT, RPT)])
    pltpu.sync_copy(zw, accw.at[pl.ds(sid * RPT, RPT)])
    plsc.subcore_barrier()
    base = sid * EPT1
    noff = cid * N_PAD
    iota = lax.iota(jnp.int32, LANES)
    zeros16 = jnp.zeros((LANES,), jnp.int32)
    ones16 = jnp.ones((LANES,), jnp.int32)

    def chunk_body(k, carry):
        off = base + k * C1
        cp1 = pltpu.async_copy(srcs.at[pl.ds(off, C1)], sbuf, sem1)
        cp2 = pltpu.async_copy(dsts.at[pl.ds(off, C1)], dbuf, sem2)
        cp1.wait()
        cp2.wait()

        def offs_body(g, c):
            sl = pl.ds(g * LANES, LANES)
            sobuf[sl] = sbuf[sl] + noff
            dobuf[sl] = dbuf[sl] + noff
            return c
        lax.fori_loop(0, GRP1, offs_body, 0)

        g1 = pltpu.async_copy(htab.at[dobuf], abuf, sem1)
        g2 = pltpu.async_copy(ltab.at[sobuf], bbuf, sem2)
        g3 = pltpu.async_copy(rtab.at[dobuf], rbuf, sem2)
        g1.wait()
        g2.wait()
        g3.wait()

        def edge_body(c, carry2):
            lrow = bbuf[c, pl.ds(0, LANES)]
            rrow = rbuf[c, pl.ds(0, LANES)]
            e = lrow + rrow
            w = jnp.exp(-jnp.maximum(e, ALPHA * e))
            s0 = w.at[zeros16].get(mode="promise_in_bounds")
            s1 = w.at[ones16].get(mode="promise_in_bounds")
            for kk in range(4):
                obuf[c, pl.ds(kk * LANES, LANES)] = (
                    abuf[c, pl.ds(kk * LANES, LANES)] * s0)
            for kk in range(4, 8):
                obuf[c, pl.ds(kk * LANES, LANES)] = (
                    abuf[c, pl.ds(kk * LANES, LANES)] * s1)
            wtbuf[c, pl.ds(0, LANES)] = jnp.where(
                iota < 2, w, jnp.zeros_like(w))
            return carry2
        lax.fori_loop(0, C1, edge_body, 0)

        pltpu.sync_copy(obuf, accd.at[sbuf], add=True)
        pltpu.sync_copy(wtbuf, accw.at[sbuf], add=True)
        return carry
    lax.fori_loop(0, NCH1, chunk_body, 0)
    plsc.subcore_barrier()
    pltpu.sync_copy(accd.at[pl.ds(sid * RPT, RPT)],
                    outd.at[pl.ds(noff + sid * RPT, RPT)])
    pltpu.sync_copy(accw.at[pl.ds(sid * RPT, RPT)],
                    outw.at[pl.ds(noff + sid * RPT, RPT)])


def _sc1(htab, ltab, rtab, srcs, dsts, zd, zw):
    mesh = plsc.VectorSubcoreMesh(core_axis_name="c", subcore_axis_name="s")
    fn = functools.partial(
        pl.kernel,
        compiler_params=pltpu.CompilerParams(use_tc_tiling_on_sc=False),
        out_type=[
            jax.ShapeDtypeStruct((NC * N_PAD, HPW), jnp.float32),
            jax.ShapeDtypeStruct((NC * N_PAD, 16), jnp.float32),
        ],
        mesh=mesh,
        scratch_types=[
            pltpu.VMEM((C1,), jnp.int32),
            pltpu.VMEM((C1,), jnp.int32),
            pltpu.VMEM((C1,), jnp.int32),
            pltpu.VMEM((C1,), jnp.int32),
            pltpu.VMEM((C1, HPW), jnp.float32),
            pltpu.VMEM((C1, 16), jnp.float32),
            pltpu.VMEM((C1, 16), jnp.float32),
            pltpu.VMEM((C1, HPW), jnp.float32),
            pltpu.VMEM((C1, 16), jnp.float32),
            pltpu.VMEM_SHARED((N_PAD, HPW), jnp.float32),
            pltpu.VMEM_SHARED((N_PAD, 16), jnp.float32),
            pltpu.SemaphoreType.DMA,
            pltpu.SemaphoreType.DMA,
        ],
    )(_sc1_body)
    return fn(htab, ltab, rtab, srcs, dsts, zd, zw)


def _tc2_body(dt_ref, db_ref, wt_ref, wb_ref, wout_ref, aor_ref, aol_ref,
              g_ref, l2_ref):
    def halves(d, wsum):
        s0 = wsum[:, 0:1] + 1e-16
        s1 = wsum[:, 1:2] + 1e-16
        return jnp.concatenate([d[:, :NHID] / s0, d[:, NHID:] / s1], axis=1)
    h2 = jnp.concatenate([halves(dt_ref[...], wt_ref[...]),
                          halves(db_ref[...], wb_ref[...])], axis=1)
    h2 = jnp.where(h2 > 0, h2, jnp.exp(h2) - 1.0)
    hout = jnp.dot(h2, wout_ref[...], preferred_element_type=jnp.float32)
    g_ref[:, :NCLASS] = hout
    g_ref[:, NCLASS:] = jnp.dot(hout, aor_ref[...],
                                preferred_element_type=jnp.float32)
    l2_ref[...] = jnp.dot(hout, aol_ref[...], preferred_element_type=jnp.float32)


def _tc2(accd, accw, Wout, AorM, AolM):
    return pl.pallas_call(
        _tc2_body,
        grid=(NBLK,),
        in_specs=[
            pl.BlockSpec((TCB, HPW), lambda i: (i, 0)),
            pl.BlockSpec((TCB, HPW), lambda i: (NBLK + i, 0)),
            pl.BlockSpec((TCB, 16), lambda i: (i, 0)),
            pl.BlockSpec((TCB, 16), lambda i: (NBLK + i, 0)),
            pl.BlockSpec((NHEADS * NHID, NCLASS), lambda i: (0, 0)),
            pl.BlockSpec((NCLASS, 16), lambda i: (0, 0)),
            pl.BlockSpec((NCLASS, 16), lambda i: (0, 0)),
        ],
        out_specs=[
            pl.BlockSpec((TCB, W2ROW), lambda i: (i, 0)),
            pl.BlockSpec((TCB, 16), lambda i: (i, 0)),
        ],
        out_shape=[
            jax.ShapeDtypeStruct((N_PAD, W2ROW), jnp.float32),
            jax.ShapeDtypeStruct((N_PAD, 16), jnp.float32),
        ],
    )(accd, accd, accw, accw, Wout, AorM, AolM)


NW = NC * NS
EPT2 = E_EDGES // NW
C2 = 80
NCH2 = EPT2 // C2


def _sc2_body(gtab, ltab2, srcs, dsts, zrows, out,
              sbuf, dbuf, abuf, bbuf, obuf, acc, sem1, sem2):
    cid = lax.axis_index("c")
    sid = lax.axis_index("s")
    wid = sid * NC + cid
    pltpu.sync_copy(zrows, acc.at[pl.ds(sid * RPT, RPT)])
    plsc.subcore_barrier()
    base = wid * EPT2
    iota = lax.iota(jnp.int32, LANES)
    zeros16 = jnp.zeros((LANES,), jnp.int32)

    def chunk_body(k, carry):
        off = base + k * C2
        cp1 = pltpu.async_copy(srcs.at[pl.ds(off, C2)], sbuf, sem1)
        cp2 = pltpu.async_copy(dsts.at[pl.ds(off, C2)], dbuf, sem2)
        cp1.wait()
        cp2.wait()
        g1 = pltpu.async_copy(gtab.at[dbuf], abuf, sem1)
        g2 = pltpu.async_copy(ltab2.at[sbuf], bbuf, sem2)
        g1.wait()
        g2.wait()

        def edge_body(c, carry2):
            lrow = bbuf[c, pl.ds(0, LANES)]
            rvec = abuf[c, pl.ds(NCLASS, LANES)]
            e = lrow + rvec
            w = jnp.exp(-jnp.maximum(e, ALPHA * e))
            s = w.at[zeros16].get(mode="promise_in_bounds")
            obuf[c, pl.ds(0, LANES)] = abuf[c, pl.ds(0, LANES)] * s
            obuf[c, pl.ds(NCLASS, LANES)] = jnp.where(
                iota < 1, w, jnp.zeros_like(w))
            return carry2
        lax.fori_loop(0, C2, edge_body, 0)

        pltpu.sync_copy(obuf, acc.at[sbuf], add=True)
        return carry
    lax.fori_loop(0, NCH2, chunk_body, 0)
    plsc.subcore_barrier()
    pltpu.sync_copy(acc.at[pl.ds(sid * RPT, RPT)],
                    out.at[pl.ds(cid * N_PAD + sid * RPT, RPT)])


def _sc2(gtab, ltab2, srcs, dsts, zrows):
    mesh = plsc.VectorSubcoreMesh(core_axis_name="c", subcore_axis_name="s")
    fn = functools.partial(
        pl.kernel,
        compiler_params=pltpu.CompilerParams(use_tc_tiling_on_sc=False),
        out_type=jax.ShapeDtypeStruct((NC * N_PAD, W2ROW), jnp.float32),
        mesh=mesh,
        scratch_types=[
            pltpu.VMEM((C2,), jnp.int32),
            pltpu.VMEM((C2,), jnp.int32),
            pltpu.VMEM((C2, W2ROW), jnp.float32),
            pltpu.VMEM((C2, 16), jnp.float32),
            pltpu.VMEM((C2, W2ROW), jnp.float32),
            pltpu.VMEM_SHARED((N_PAD, W2ROW), jnp.float32),
            pltpu.SemaphoreType.DMA,
            pltpu.SemaphoreType.DMA,
        ],
    )(_sc2_body)
    return fn(gtab, ltab2, srcs, dsts, zrows)


def _tc3_body(top_ref, bot_ref, out_ref):
    s = top_ref[...] + bot_ref[...]
    hp = s[:, :NCLASS] / (s[:, NCLASS:NCLASS + 1] + 1e-16)
    o = jnp.where(hp > 0, hp, jnp.exp(hp) - 1.0)
    m = jnp.max(o, axis=1, keepdims=True)
    lse = jnp.log(jnp.sum(jnp.exp(o - m), axis=1, keepdims=True)) + m
    out_ref[...] = o - lse


def _tc3(acc2):
    return pl.pallas_call(
        _tc3_body,
        grid=(NBLK,),
        in_specs=[
            pl.BlockSpec((TCB, W2ROW), lambda i: (i, 0)),
            pl.BlockSpec((TCB, W2ROW), lambda i: (NBLK + i, 0)),
        ],
        out_specs=pl.BlockSpec((TCB, NCLASS), lambda i: (i, 0)),
        out_shape=jax.ShapeDtypeStruct((N_PAD, NCLASS), jnp.float32),
    )(acc2, acc2)


def kernel(x, adj, W0, a0, W1, a1, W2, a2, W3, a3, W_out, a_out):
    xp = jnp.pad(x, ((0, N_PAD - N_NODES), (0, 0)))
    src = adj[0]
    dst = adj[1]

    Wc = jnp.stack([jnp.concatenate([W0, W1], axis=1),
                    jnp.concatenate([W2, W3], axis=1)])
    als = [a[0, :NHID] for a in (a0, a1, a2, a3)]
    ars = [a[0, NHID:] for a in (a0, a1, a2, a3)]
    ALm = jnp.zeros((NC, HPW, 16), jnp.float32)
    ERm = jnp.zeros((NC, HPW, 16), jnp.float32)
    for c in range(NC):
        ALm = ALm.at[c, :NHID, 0].set(als[2 * c])
        ALm = ALm.at[c, NHID:, 1].set(als[2 * c + 1])
        ERm = ERm.at[c, :NHID, 0].set(ars[2 * c])
        ERm = ERm.at[c, NHID:, 1].set(ars[2 * c + 1])
    AolM = jnp.zeros((NCLASS, 16), jnp.float32).at[:, 0].set(a_out[0, :NCLASS])
    AorM = jnp.zeros((NCLASS, 16), jnp.float32).at[:, 0].set(a_out[0, NCLASS:])

    zd = jnp.zeros((RPT, HPW), jnp.float32)
    zw = jnp.zeros((RPT, 16), jnp.float32)
    zr2 = jnp.zeros((RPT, W2ROW), jnp.float32)

    htab, ltab, rtab = _tc1(xp, Wc, ALm, ERm)
    accd, accw = _sc1(htab, ltab, rtab, src, dst, zd, zw)
    gtab, ltab2 = _tc2(accd, accw, W_out, AorM, AolM)
    acc2 = _sc2(gtab, ltab2, src, dst, zr2)
    outp = _tc3(acc2)
    return outp[:N_NODES]


# speedup vs baseline: 18.9637x; 3.6551x over previous
import functools

import jax
import jax.numpy as jnp
from jax import lax
from jax.experimental import pallas as pl
from jax.experimental.pallas import tpu as pltpu
from jax.experimental.pallas import tpu_sc as plsc

N_NODES = 10000
N_PAD = 10240
E_EDGES = 320000
NFEAT = 128
NHID = 64
NHEADS = 4
NCLASS = 16
ALPHA = 0.2

NC = 2
NS = 16
LANES = 16

HPW = 2 * NHID
W2ROW = 2 * NCLASS

TCB = 1024
NBLK = N_PAD // TCB

CH = 80


def _tc1_body(x_ref, wc_ref, lrm_ref, h_ref, lrtab_ref):
    h = jnp.dot(x_ref[...], wc_ref[0], preferred_element_type=jnp.float32)
    h_ref[...] = h
    lrtab_ref[...] = jnp.dot(h, lrm_ref[0], preferred_element_type=jnp.float32)


def _tc1(xp, Wc, LRm):
    return pl.pallas_call(
        _tc1_body,
        grid=(NC, NBLK),
        in_specs=[
            pl.BlockSpec((TCB, NFEAT), lambda c, i: (i, 0)),
            pl.BlockSpec((1, NFEAT, HPW), lambda c, i: (c, 0, 0)),
            pl.BlockSpec((1, HPW, 16), lambda c, i: (c, 0, 0)),
        ],
        out_specs=[
            pl.BlockSpec((TCB, HPW), lambda c, i: (c * NBLK + i, 0)),
            pl.BlockSpec((TCB, 16), lambda c, i: (c * NBLK + i, 0)),
        ],
        out_shape=[
            jax.ShapeDtypeStruct((NC * N_PAD, HPW), jnp.float32),
            jax.ShapeDtypeStruct((NC * N_PAD, 16), jnp.float32),
        ],
    )(xp, Wc, LRm)


EPT1 = E_EDGES // NS
NCH1 = EPT1 // CH
GRP = CH // LANES
N_ACC = 10112
RPT = N_ACC // NS


def _sc1_body(htab, lrtab, srcs, dsts, zd, zw, outd, outw,
              sct, obuf, wtbuf,
              sbuf0, dbuf0, sobuf0, abuf0, bbuf0, rbuf0,
              sbuf1, dbuf1, sobuf1, abuf1, bbuf1, rbuf1,
              accd, accw, semg0, semg1, semi0, semi1):
    cid = lax.axis_index("c")
    sid = lax.axis_index("s")
    pltpu.sync_copy(zd, accd.at[pl.ds(sid * RPT, RPT)])
    pltpu.sync_copy(zw, accw.at[pl.ds(sid * RPT, RPT)])
    noff = cid * N_PAD
    iota = lax.iota(jnp.int32, LANES)
    zeros16 = jnp.zeros((LANES,), jnp.int32)
    ones16 = jnp.ones((LANES,), jnp.int32)
    shidx = jnp.bitwise_and(iota + 2, 15)
    sbufs = (sbuf0, sbuf1)
    dbufs = (dbuf0, dbuf1)
    sobufs = (sobuf0, sobuf1)
    abufs = (abuf0, abuf1)
    bbufs = (bbuf0, bbuf1)
    rbufs = (rbuf0, rbuf1)
    semgs = (semg0, semg1)
    semis = (semi0, semi1)
    eb = sid * EPT1
    plsc.subcore_barrier()

    def issue_idx(k, b):
        sl = pl.ds(eb + k * CH, CH)
        pltpu.async_copy(srcs.at[sl], sbufs[b], semis[b])
        pltpu.async_copy(dsts.at[sl], dbufs[b], semis[b])

    def wait_idx(k, b):
        sl = pl.ds(eb + k * CH, CH)
        pltpu.make_async_copy(srcs.at[sl], sbufs[b], semis[b]).wait()
        pltpu.make_async_copy(dsts.at[sl], dbufs[b], semis[b]).wait()

    def offsets(b):
        for g in range(GRP):
            sl = pl.ds(g * LANES, LANES)
            sobufs[b][sl] = sbufs[b][sl] + noff
            dbufs[b][sl] = dbufs[b][sl] + noff

    def issue_gather(b):
        pltpu.async_copy(htab.at[dbufs[b]], abufs[b], semgs[b])
        pltpu.async_copy(lrtab.at[sobufs[b]], bbufs[b], semgs[b])
        pltpu.async_copy(lrtab.at[dbufs[b]], rbufs[b], semgs[b])

    def wait_gather(b):
        pltpu.make_async_copy(htab.at[dbufs[b]], abufs[b], semgs[b]).wait()
        pltpu.make_async_copy(lrtab.at[sobufs[b]], bbufs[b], semgs[b]).wait()
        pltpu.make_async_copy(lrtab.at[dbufs[b]], rbufs[b], semgs[b]).wait()

    def compute(b):
        abuf, bbuf, rbuf = abufs[b], bbufs[b], rbufs[b]

        @plsc.parallel_loop(0, CH, 1, unroll=4)
        def _(c):
            lrow = bbuf[c, pl.ds(0, LANES)]
            rrow = rbuf[c, pl.ds(0, LANES)]
            rsh = rrow.at[shidx].get(mode="promise_in_bounds")
            e = lrow + rsh
            w = jnp.exp(-jnp.maximum(e, ALPHA * e))
            s0 = w.at[zeros16].get(mode="promise_in_bounds")
            s1 = w.at[ones16].get(mode="promise_in_bounds")
            for kk in range(4):
                obuf[c, pl.ds(kk * LANES, LANES)] = (
                    abuf[c, pl.ds(kk * LANES, LANES)] * s0)
            for kk in range(4, 8):
                obuf[c, pl.ds(kk * LANES, LANES)] = (
                    abuf[c, pl.ds(kk * LANES, LANES)] * s1)
            wtbuf[c, pl.ds(0, LANES)] = jnp.where(
                iota < 2, w, jnp.zeros_like(w))

    def scatter(b):
        pltpu.sync_copy(obuf, accd.at[sct], add=True)
        pltpu.sync_copy(wtbuf, accw.at[sct], add=True)

    issue_idx(0, 0)
    issue_idx(1, 1)
    wait_idx(0, 0)
    offsets(0)
    issue_gather(0)

    def pair_body(kk, carry):
        for b in range(2):
            k = 2 * kk + b
            bn = 1 - b
            if b == 0:
                wait_idx(k + 1, bn)
                offsets(bn)
                issue_gather(bn)
            else:
                @pl.when(kk < NCH1 // 2 - 1)
                def _():
                    wait_idx(k + 1, bn)
                    offsets(bn)
                    issue_gather(bn)
            wait_gather(b)
            for g in range(GRP):
                sl = pl.ds(g * LANES, LANES)
                sct[sl] = sbufs[b][sl]

            @pl.when(k + 2 < NCH1)
            def _():
                issue_idx(k + 2, b)
            compute(b)
            scatter(b)
        return carry
    lax.fori_loop(0, NCH1 // 2, pair_body, 0)
    plsc.subcore_barrier()
    pltpu.sync_copy(accd.at[pl.ds(sid * RPT, RPT)],
                    outd.at[pl.ds(cid * N_ACC + sid * RPT, RPT)])
    pltpu.sync_copy(accw.at[pl.ds(sid * RPT, RPT)],
                    outw.at[pl.ds(cid * N_ACC + sid * RPT, RPT)])


def _sc1(htab, lrtab, srcs, dsts, zd, zw):
    mesh = plsc.VectorSubcoreMesh(core_axis_name="c", subcore_axis_name="s")
    fn = functools.partial(
        pl.kernel,
        compiler_params=pltpu.CompilerParams(use_tc_tiling_on_sc=False),
        out_type=[
            jax.ShapeDtypeStruct((NC * N_ACC, HPW), jnp.float32),
            jax.ShapeDtypeStruct((NC * N_ACC, 16), jnp.float32),
        ],
        mesh=mesh,
        scratch_types=[
            pltpu.VMEM((CH,), jnp.int32),
            pltpu.VMEM((CH, HPW), jnp.float32),
            pltpu.VMEM((CH, 16), jnp.float32),
            pltpu.VMEM((CH,), jnp.int32),
            pltpu.VMEM((CH,), jnp.int32),
            pltpu.VMEM((CH,), jnp.int32),
            pltpu.VMEM((CH, HPW), jnp.float32),
            pltpu.VMEM((CH, 16), jnp.float32),
            pltpu.VMEM((CH, 16), jnp.float32),
            pltpu.VMEM((CH,), jnp.int32),
            pltpu.VMEM((CH,), jnp.int32),
            pltpu.VMEM((CH,), jnp.int32),
            pltpu.VMEM((CH, HPW), jnp.float32),
            pltpu.VMEM((CH, 16), jnp.float32),
            pltpu.VMEM((CH, 16), jnp.float32),
            pltpu.VMEM_SHARED((N_ACC, HPW), jnp.float32),
            pltpu.VMEM_SHARED((N_ACC, 16), jnp.float32),
            pltpu.SemaphoreType.DMA,
            pltpu.SemaphoreType.DMA,
            pltpu.SemaphoreType.DMA,
            pltpu.SemaphoreType.DMA,
        ],
    )(_sc1_body)
    return fn(htab, lrtab, srcs, dsts, zd, zw)


def _tc2_body(dt_ref, db_ref, wt_ref, wb_ref, wout_ref, aor_ref, aol_ref,
              g_ref, l2_ref):
    def halves(d, wsum):
        s0 = wsum[:, 0:1] + 1e-16
        s1 = wsum[:, 1:2] + 1e-16
        return jnp.concatenate([d[:, :NHID] / s0, d[:, NHID:] / s1], axis=1)
    h2 = jnp.concatenate([halves(dt_ref[...], wt_ref[...]),
                          halves(db_ref[...], wb_ref[...])], axis=1)
    h2 = jnp.where(h2 > 0, h2, jnp.exp(h2) - 1.0)
    hout = jnp.dot(h2, wout_ref[...], preferred_element_type=jnp.float32)
    g_ref[:, :NCLASS] = hout
    g_ref[:, NCLASS:] = jnp.dot(hout, aor_ref[...],
                                preferred_element_type=jnp.float32)
    l2_ref[...] = jnp.dot(hout, aol_ref[...], preferred_element_type=jnp.float32)


def _tc2(accd, accw, Wout, AorM, AolM):
    return pl.pallas_call(
        _tc2_body,
        grid=(NS,),
        in_specs=[
            pl.BlockSpec((RPT, HPW), lambda i: (i, 0)),
            pl.BlockSpec((RPT, HPW), lambda i: (NS + i, 0)),
            pl.BlockSpec((RPT, 16), lambda i: (i, 0)),
            pl.BlockSpec((RPT, 16), lambda i: (NS + i, 0)),
            pl.BlockSpec((NHEADS * NHID, NCLASS), lambda i: (0, 0)),
            pl.BlockSpec((NCLASS, 16), lambda i: (0, 0)),
            pl.BlockSpec((NCLASS, 16), lambda i: (0, 0)),
        ],
        out_specs=[
            pl.BlockSpec((RPT, W2ROW), lambda i: (i, 0)),
            pl.BlockSpec((RPT, 16), lambda i: (i, 0)),
        ],
        out_shape=[
            jax.ShapeDtypeStruct((N_ACC, W2ROW), jnp.float32),
            jax.ShapeDtypeStruct((N_ACC, 16), jnp.float32),
        ],
    )(accd, accd, accw, accw, Wout, AorM, AolM)


NW = NC * NS
EPT2 = E_EDGES // NW
NCH2 = EPT2 // CH


def _sc2_body(gtab, ltab2, srcs, dsts, zrows, out,
              sct, obuf,
              sbuf0, dbuf0, abuf0, bbuf0,
              sbuf1, dbuf1, abuf1, bbuf1,
              acc, semg0, semg1, semi0, semi1):
    cid = lax.axis_index("c")
    sid = lax.axis_index("s")
    wid = sid * NC + cid
    pltpu.sync_copy(zrows, acc.at[pl.ds(sid * RPT, RPT)])
    iota = lax.iota(jnp.int32, LANES)
    zeros16 = jnp.zeros((LANES,), jnp.int32)
    sbufs = (sbuf0, sbuf1)
    dbufs = (dbuf0, dbuf1)
    abufs = (abuf0, abuf1)
    bbufs = (bbuf0, bbuf1)
    semgs = (semg0, semg1)
    semis = (semi0, semi1)
    eb = wid * EPT2
    plsc.subcore_barrier()

    def issue_idx(k, b):
        sl = pl.ds(eb + k * CH, CH)
        pltpu.async_copy(srcs.at[sl], sbufs[b], semis[b])
        pltpu.async_copy(dsts.at[sl], dbufs[b], semis[b])

    def wait_idx(k, b):
        sl = pl.ds(eb + k * CH, CH)
        pltpu.make_async_copy(srcs.at[sl], sbufs[b], semis[b]).wait()
        pltpu.make_async_copy(dsts.at[sl], dbufs[b], semis[b]).wait()

    def issue_gather(b):
        pltpu.async_copy(gtab.at[dbufs[b]], abufs[b], semgs[b])
        pltpu.async_copy(ltab2.at[sbufs[b]], bbufs[b], semgs[b])

    def wait_gather(b):
        pltpu.make_async_copy(gtab.at[dbufs[b]], abufs[b], semgs[b]).wait()
        pltpu.make_async_copy(ltab2.at[sbufs[b]], bbufs[b], semgs[b]).wait()

    def compute(b):
        abuf, bbuf = abufs[b], bbufs[b]

        @plsc.parallel_loop(0, CH, 1, unroll=4)
        def _(c):
            lrow = bbuf[c, pl.ds(0, LANES)]
            rvec = abuf[c, pl.ds(NCLASS, LANES)]
            e = lrow + rvec
            w = jnp.exp(-jnp.maximum(e, ALPHA * e))
            s = w.at[zeros16].get(mode="promise_in_bounds")
            obuf[c, pl.ds(0, LANES)] = abuf[c, pl.ds(0, LANES)] * s
            obuf[c, pl.ds(NCLASS, LANES)] = jnp.where(
                iota < 1, w, jnp.zeros_like(w))

    def scatter(b):
        pltpu.sync_copy(obuf, acc.at[sct], add=True)

    issue_idx(0, 0)
    issue_idx(1, 1)
    wait_idx(0, 0)
    issue_gather(0)

    def pair_body(kk, carry):
        for b in range(2):
            k = 2 * kk + b
            bn = 1 - b

            @pl.when(k + 1 < NCH2)
            def _():
                wait_idx(k + 1, bn)
                issue_gather(bn)

            @pl.when(k < NCH2)
            def _():
                wait_gather(b)
                for g in range(GRP):
                    sl = pl.ds(g * LANES, LANES)
                    sct[sl] = sbufs[b][sl]

            @pl.when(k + 2 < NCH2)
            def _():
                issue_idx(k + 2, b)

            @pl.when(k < NCH2)
            def _():
                compute(b)
                scatter(b)
        return carry
    lax.fori_loop(0, (NCH2 + 1) // 2, pair_body, 0)
    plsc.subcore_barrier()
    pltpu.sync_copy(acc.at[pl.ds(sid * RPT, RPT)],
                    out.at[pl.ds(cid * N_ACC + sid * RPT, RPT)])


def _sc2(gtab, ltab2, srcs, dsts, zrows):
    mesh = plsc.VectorSubcoreMesh(core_axis_name="c", subcore_axis_name="s")
    fn = functools.partial(
        pl.kernel,
        compiler_params=pltpu.CompilerParams(use_tc_tiling_on_sc=False),
        out_type=jax.ShapeDtypeStruct((NC * N_ACC, W2ROW), jnp.float32),
        mesh=mesh,
        scratch_types=[
            pltpu.VMEM((CH,), jnp.int32),
            pltpu.VMEM((CH, W2ROW), jnp.float32),
            pltpu.VMEM((CH,), jnp.int32),
            pltpu.VMEM((CH,), jnp.int32),
            pltpu.VMEM((CH, W2ROW), jnp.float32),
            pltpu.VMEM((CH, 16), jnp.float32),
            pltpu.VMEM((CH,), jnp.int32),
            pltpu.VMEM((CH,), jnp.int32),
            pltpu.VMEM((CH, W2ROW), jnp.float32),
            pltpu.VMEM((CH, 16), jnp.float32),
            pltpu.VMEM_SHARED((N_ACC, W2ROW), jnp.float32),
            pltpu.SemaphoreType.DMA,
            pltpu.SemaphoreType.DMA,
            pltpu.SemaphoreType.DMA,
            pltpu.SemaphoreType.DMA,
        ],
    )(_sc2_body)
    return fn(gtab, ltab2, srcs, dsts, zrows)


def _tc3_body(top_ref, bot_ref, out_ref):
    s = top_ref[...] + bot_ref[...]
    hp = s[:, :NCLASS] / (s[:, NCLASS:NCLASS + 1] + 1e-16)
    o = jnp.where(hp > 0, hp, jnp.exp(hp) - 1.0)
    m = jnp.max(o, axis=1, keepdims=True)
    lse = jnp.log(jnp.sum(jnp.exp(o - m), axis=1, keepdims=True)) + m
    out_ref[...] = o - lse


def _tc3(acc2):
    return pl.pallas_call(
        _tc3_body,
        grid=(NS,),
        in_specs=[
            pl.BlockSpec((RPT, W2ROW), lambda i: (i, 0)),
            pl.BlockSpec((RPT, W2ROW), lambda i: (NS + i, 0)),
        ],
        out_specs=pl.BlockSpec((RPT, NCLASS), lambda i: (i, 0)),
        out_shape=jax.ShapeDtypeStruct((N_ACC, NCLASS), jnp.float32),
    )(acc2, acc2)


def kernel(x, adj, W0, a0, W1, a1, W2, a2, W3, a3, W_out, a_out):
    xp = jnp.pad(x, ((0, N_PAD - N_NODES), (0, 0)))
    src = adj[0]
    dst = adj[1]

    Wc = jnp.stack([jnp.concatenate([W0, W1], axis=1),
                    jnp.concatenate([W2, W3], axis=1)])
    als = [a[0, :NHID] for a in (a0, a1, a2, a3)]
    ars = [a[0, NHID:] for a in (a0, a1, a2, a3)]
    LRm = jnp.zeros((NC, HPW, 16), jnp.float32)
    for c in range(NC):
        LRm = LRm.at[c, :NHID, 0].set(als[2 * c])
        LRm = LRm.at[c, NHID:, 1].set(als[2 * c + 1])
        LRm = LRm.at[c, :NHID, 2].set(ars[2 * c])
        LRm = LRm.at[c, NHID:, 3].set(ars[2 * c + 1])
    AolM = jnp.zeros((NCLASS, 16), jnp.float32).at[:, 0].set(a_out[0, :NCLASS])
    AorM = jnp.zeros((NCLASS, 16), jnp.float32).at[:, 0].set(a_out[0, NCLASS:])

    zd = jnp.zeros((RPT, HPW), jnp.float32)
    zw = jnp.zeros((RPT, 16), jnp.float32)
    zr2 = jnp.zeros((RPT, W2ROW), jnp.float32)

    htab, lrtab = _tc1(xp, Wc, LRm)
    accd, accw = _sc1(htab, lrtab, src, dst, zd, zw)
    gtab, ltab2 = _tc2(accd, accw, W_out, AorM, AolM)
    acc2 = _sc2(gtab, ltab2, src, dst, zr2)
    outp = _tc3(acc2)
    return outp[:N_NODES]


# speedup vs baseline: 19.1061x; 1.0075x over previous
import functools

import jax
import jax.numpy as jnp
from jax import lax
from jax.experimental import pallas as pl
from jax.experimental.pallas import tpu as pltpu
from jax.experimental.pallas import tpu_sc as plsc

N_NODES = 10000
N_PAD = 10240
E_EDGES = 320000
NFEAT = 128
NHID = 64
NHEADS = 4
NCLASS = 16
ALPHA = 0.2

NC = 2
NS = 16
LANES = 16

HPW = 2 * NHID
W2ROW = 2 * NCLASS

TCB = 1024
NBLK = N_PAD // TCB

CH = 80


def _tc1_body(x_ref, wc_ref, lrm_ref, h_ref, lrtab_ref):
    h = jnp.dot(x_ref[...], wc_ref[0], preferred_element_type=jnp.float32)
    h_ref[...] = h
    lrtab_ref[...] = jnp.dot(h, lrm_ref[0], preferred_element_type=jnp.float32)


def _tc1(xp, Wc, LRm):
    return pl.pallas_call(
        _tc1_body,
        grid=(NC, NBLK),
        in_specs=[
            pl.BlockSpec((TCB, NFEAT), lambda c, i: (i, 0)),
            pl.BlockSpec((1, NFEAT, HPW), lambda c, i: (c, 0, 0)),
            pl.BlockSpec((1, HPW, 16), lambda c, i: (c, 0, 0)),
        ],
        out_specs=[
            pl.BlockSpec((TCB, HPW), lambda c, i: (c * NBLK + i, 0)),
            pl.BlockSpec((TCB, 16), lambda c, i: (c * NBLK + i, 0)),
        ],
        out_shape=[
            jax.ShapeDtypeStruct((NC * N_PAD, HPW), jnp.float32),
            jax.ShapeDtypeStruct((NC * N_PAD, 16), jnp.float32),
        ],
    )(xp, Wc, LRm)


EPT1 = E_EDGES // NS
NCH1 = EPT1 // CH
GRP = CH // LANES
N_ACC = 10112
RPT = N_ACC // NS


def _sc1_body(htab, lrtab, srcs, dsts, zd, zw, outd, outw,
              sct, obuf, wtbuf,
              sbuf0, dbuf0, sobuf0, abuf0, bbuf0, rbuf0,
              sbuf1, dbuf1, sobuf1, abuf1, bbuf1, rbuf1,
              accd, accw, semg0, semg1, semi0, semi1):
    cid = lax.axis_index("c")
    sid = lax.axis_index("s")
    pltpu.sync_copy(zd, accd.at[pl.ds(sid * RPT, RPT)])
    pltpu.sync_copy(zw, accw.at[pl.ds(sid * RPT, RPT)])
    noff = cid * N_PAD
    iota = lax.iota(jnp.int32, LANES)
    zeros16 = jnp.zeros((LANES,), jnp.int32)
    ones16 = jnp.ones((LANES,), jnp.int32)
    shidx = jnp.bitwise_and(iota + 2, 15)
    sbufs = (sbuf0, sbuf1)
    dbufs = (dbuf0, dbuf1)
    sobufs = (sobuf0, sobuf1)
    abufs = (abuf0, abuf1)
    bbufs = (bbuf0, bbuf1)
    rbufs = (rbuf0, rbuf1)
    semgs = (semg0, semg1)
    semis = (semi0, semi1)
    eb = sid * EPT1
    plsc.subcore_barrier()

    def issue_idx(k, b):
        sl = pl.ds(eb + k * CH, CH)
        pltpu.async_copy(srcs.at[sl], sbufs[b], semis[b])
        pltpu.async_copy(dsts.at[sl], dbufs[b], semis[b])

    def wait_idx(k, b):
        sl = pl.ds(eb + k * CH, CH)
        pltpu.make_async_copy(srcs.at[sl], sbufs[b], semis[b]).wait()
        pltpu.make_async_copy(dsts.at[sl], dbufs[b], semis[b]).wait()

    def offsets(b):
        for g in range(GRP):
            sl = pl.ds(g * LANES, LANES)
            sobufs[b][sl] = sbufs[b][sl] + noff
            dbufs[b][sl] = dbufs[b][sl] + noff

    def issue_gather(b):
        pltpu.async_copy(htab.at[dbufs[b]], abufs[b], semgs[b])
        pltpu.async_copy(lrtab.at[sobufs[b]], bbufs[b], semgs[b])
        pltpu.async_copy(lrtab.at[dbufs[b]], rbufs[b], semgs[b])

    def wait_gather(b):
        pltpu.make_async_copy(htab.at[dbufs[b]], abufs[b], semgs[b]).wait()
        pltpu.make_async_copy(lrtab.at[sobufs[b]], bbufs[b], semgs[b]).wait()
        pltpu.make_async_copy(lrtab.at[dbufs[b]], rbufs[b], semgs[b]).wait()

    def compute(b):
        abuf, bbuf, rbuf = abufs[b], bbufs[b], rbufs[b]

        @plsc.parallel_loop(0, CH, 1, unroll=2)
        def _(c):
            lrow = bbuf[c, pl.ds(0, LANES)]
            rrow = rbuf[c, pl.ds(0, LANES)]
            rsh = rrow.at[shidx].get(mode="promise_in_bounds")
            e = lrow + rsh
            w = jnp.exp(-jnp.maximum(e, ALPHA * e))
            s0 = w.at[zeros16].get(mode="promise_in_bounds")
            s1 = w.at[ones16].get(mode="promise_in_bounds")
            for kk in range(4):
                obuf[c, pl.ds(kk * LANES, LANES)] = (
                    abuf[c, pl.ds(kk * LANES, LANES)] * s0)
            for kk in range(4, 8):
                obuf[c, pl.ds(kk * LANES, LANES)] = (
                    abuf[c, pl.ds(kk * LANES, LANES)] * s1)
            wtbuf[c, pl.ds(0, LANES)] = jnp.where(
                iota < 2, w, jnp.zeros_like(w))

    def scatter(b):
        pltpu.sync_copy(obuf, accd.at[sct], add=True)
        pltpu.sync_copy(wtbuf, accw.at[sct], add=True)

    issue_idx(0, 0)
    issue_idx(1, 1)
    wait_idx(0, 0)
    offsets(0)
    issue_gather(0)

    def pair_body(kk, carry):
        for b in range(2):
            k = 2 * kk + b
            bn = 1 - b
            if b == 0:
                wait_idx(k + 1, bn)
                offsets(bn)
                issue_gather(bn)
            else:
                @pl.when(kk < NCH1 // 2 - 1)
                def _():
                    wait_idx(k + 1, bn)
                    offsets(bn)
                    issue_gather(bn)
            wait_gather(b)
            for g in range(GRP):
                sl = pl.ds(g * LANES, LANES)
                sct[sl] = sbufs[b][sl]

            @pl.when(k + 2 < NCH1)
            def _():
                issue_idx(k + 2, b)
            compute(b)
            scatter(b)
        return carry
    lax.fori_loop(0, NCH1 // 2, pair_body, 0)
    plsc.subcore_barrier()
    pltpu.sync_copy(accd.at[pl.ds(sid * RPT, RPT)],
                    outd.at[pl.ds(cid * N_ACC + sid * RPT, RPT)])
    pltpu.sync_copy(accw.at[pl.ds(sid * RPT, RPT)],
                    outw.at[pl.ds(cid * N_ACC + sid * RPT, RPT)])


def _sc1(htab, lrtab, srcs, dsts, zd, zw):
    mesh = plsc.VectorSubcoreMesh(core_axis_name="c", subcore_axis_name="s")
    fn = functools.partial(
        pl.kernel,
        compiler_params=pltpu.CompilerParams(use_tc_tiling_on_sc=False),
        out_type=[
            jax.ShapeDtypeStruct((NC * N_ACC, HPW), jnp.float32),
            jax.ShapeDtypeStruct((NC * N_ACC, 16), jnp.float32),
        ],
        mesh=mesh,
        scratch_types=[
            pltpu.VMEM((CH,), jnp.int32),
            pltpu.VMEM((CH, HPW), jnp.float32),
            pltpu.VMEM((CH, 16), jnp.float32),
            pltpu.VMEM((CH,), jnp.int32),
            pltpu.VMEM((CH,), jnp.int32),
            pltpu.VMEM((CH,), jnp.int32),
            pltpu.VMEM((CH, HPW), jnp.float32),
            pltpu.VMEM((CH, 16), jnp.float32),
            pltpu.VMEM((CH, 16), jnp.float32),
            pltpu.VMEM((CH,), jnp.int32),
            pltpu.VMEM((CH,), jnp.int32),
            pltpu.VMEM((CH,), jnp.int32),
            pltpu.VMEM((CH, HPW), jnp.float32),
            pltpu.VMEM((CH, 16), jnp.float32),
            pltpu.VMEM((CH, 16), jnp.float32),
            pltpu.VMEM_SHARED((N_ACC, HPW), jnp.float32),
            pltpu.VMEM_SHARED((N_ACC, 16), jnp.float32),
            pltpu.SemaphoreType.DMA,
            pltpu.SemaphoreType.DMA,
            pltpu.SemaphoreType.DMA,
            pltpu.SemaphoreType.DMA,
        ],
    )(_sc1_body)
    return fn(htab, lrtab, srcs, dsts, zd, zw)


def _tc2_body(dt_ref, db_ref, wt_ref, wb_ref, wout_ref, aor_ref, aol_ref,
              g_ref, l2_ref):
    def halves(d, wsum):
        s0 = wsum[:, 0:1] + 1e-16
        s1 = wsum[:, 1:2] + 1e-16
        return jnp.concatenate([d[:, :NHID] / s0, d[:, NHID:] / s1], axis=1)
    h2 = jnp.concatenate([halves(dt_ref[...], wt_ref[...]),
                          halves(db_ref[...], wb_ref[...])], axis=1)
    h2 = jnp.where(h2 > 0, h2, jnp.exp(h2) - 1.0)
    hout = jnp.dot(h2, wout_ref[...], preferred_element_type=jnp.float32)
    g_ref[:, :NCLASS] = hout
    g_ref[:, NCLASS:] = jnp.dot(hout, aor_ref[...],
                                preferred_element_type=jnp.float32)
    l2_ref[...] = jnp.dot(hout, aol_ref[...], preferred_element_type=jnp.float32)


def _tc2(accd, accw, Wout, AorM, AolM):
    return pl.pallas_call(
        _tc2_body,
        grid=(NS,),
        in_specs=[
            pl.BlockSpec((RPT, HPW), lambda i: (i, 0)),
            pl.BlockSpec((RPT, HPW), lambda i: (NS + i, 0)),
            pl.BlockSpec((RPT, 16), lambda i: (i, 0)),
            pl.BlockSpec((RPT, 16), lambda i: (NS + i, 0)),
            pl.BlockSpec((NHEADS * NHID, NCLASS), lambda i: (0, 0)),
            pl.BlockSpec((NCLASS, 16), lambda i: (0, 0)),
            pl.BlockSpec((NCLASS, 16), lambda i: (0, 0)),
        ],
        out_specs=[
            pl.BlockSpec((RPT, W2ROW), lambda i: (i, 0)),
            pl.BlockSpec((RPT, 16), lambda i: (i, 0)),
        ],
        out_shape=[
            jax.ShapeDtypeStruct((N_ACC, W2ROW), jnp.float32),
            jax.ShapeDtypeStruct((N_ACC, 16), jnp.float32),
        ],
    )(accd, accd, accw, accw, Wout, AorM, AolM)


NW = NC * NS
EPT2 = E_EDGES // NW
NCH2 = EPT2 // CH


def _sc2_body(gtab, ltab2, srcs, dsts, zrows, out,
              sct, obuf,
              sbuf0, dbuf0, abuf0, bbuf0,
              sbuf1, dbuf1, abuf1, bbuf1,
              acc, semg0, semg1, semi0, semi1):
    cid = lax.axis_index("c")
    sid = lax.axis_index("s")
    wid = sid * NC + cid
    pltpu.sync_copy(zrows, acc.at[pl.ds(sid * RPT, RPT)])
    iota = lax.iota(jnp.int32, LANES)
    zeros16 = jnp.zeros((LANES,), jnp.int32)
    sbufs = (sbuf0, sbuf1)
    dbufs = (dbuf0, dbuf1)
    abufs = (abuf0, abuf1)
    bbufs = (bbuf0, bbuf1)
    semgs = (semg0, semg1)
    semis = (semi0, semi1)
    eb = wid * EPT2
    plsc.subcore_barrier()

    def issue_idx(k, b):
        sl = pl.ds(eb + k * CH, CH)
        pltpu.async_copy(srcs.at[sl], sbufs[b], semis[b])
        pltpu.async_copy(dsts.at[sl], dbufs[b], semis[b])

    def wait_idx(k, b):
        sl = pl.ds(eb + k * CH, CH)
        pltpu.make_async_copy(srcs.at[sl], sbufs[b], semis[b]).wait()
        pltpu.make_async_copy(dsts.at[sl], dbufs[b], semis[b]).wait()

    def issue_gather(b):
        pltpu.async_copy(gtab.at[dbufs[b]], abufs[b], semgs[b])
        pltpu.async_copy(ltab2.at[sbufs[b]], bbufs[b], semgs[b])

    def wait_gather(b):
        pltpu.make_async_copy(gtab.at[dbufs[b]], abufs[b], semgs[b]).wait()
        pltpu.make_async_copy(ltab2.at[sbufs[b]], bbufs[b], semgs[b]).wait()

    def compute(b):
        abuf, bbuf = abufs[b], bbufs[b]

        @plsc.parallel_loop(0, CH, 1, unroll=2)
        def _(c):
            lrow = bbuf[c, pl.ds(0, LANES)]
            rvec = abuf[c, pl.ds(NCLASS, LANES)]
            e = lrow + rvec
            w = jnp.exp(-jnp.maximum(e, ALPHA * e))
            s = w.at[zeros16].get(mode="promise_in_bounds")
            obuf[c, pl.ds(0, LANES)] = abuf[c, pl.ds(0, LANES)] * s
            obuf[c, pl.ds(NCLASS, LANES)] = jnp.where(
                iota < 1, w, jnp.zeros_like(w))

    def scatter(b):
        pltpu.sync_copy(obuf, acc.at[sct], add=True)

    issue_idx(0, 0)
    issue_idx(1, 1)
    wait_idx(0, 0)
    issue_gather(0)

    def pair_body(kk, carry):
        for b in range(2):
            k = 2 * kk + b
            bn = 1 - b

            @pl.when(k + 1 < NCH2)
            def _():
                wait_idx(k + 1, bn)
                issue_gather(bn)

            @pl.when(k < NCH2)
            def _():
                wait_gather(b)
                for g in range(GRP):
                    sl = pl.ds(g * LANES, LANES)
                    sct[sl] = sbufs[b][sl]

            @pl.when(k + 2 < NCH2)
            def _():
                issue_idx(k + 2, b)

            @pl.when(k < NCH2)
            def _():
                compute(b)
                scatter(b)
        return carry
    lax.fori_loop(0, (NCH2 + 1) // 2, pair_body, 0)
    plsc.subcore_barrier()
    pltpu.sync_copy(acc.at[pl.ds(sid * RPT, RPT)],
                    out.at[pl.ds(cid * N_ACC + sid * RPT, RPT)])


def _sc2(gtab, ltab2, srcs, dsts, zrows):
    mesh = plsc.VectorSubcoreMesh(core_axis_name="c", subcore_axis_name="s")
    fn = functools.partial(
        pl.kernel,
        compiler_params=pltpu.CompilerParams(use_tc_tiling_on_sc=False),
        out_type=jax.ShapeDtypeStruct((NC * N_ACC, W2ROW), jnp.float32),
        mesh=mesh,
        scratch_types=[
            pltpu.VMEM((CH,), jnp.int32),
            pltpu.VMEM((CH, W2ROW), jnp.float32),
            pltpu.VMEM((CH,), jnp.int32),
            pltpu.VMEM((CH,), jnp.int32),
            pltpu.VMEM((CH, W2ROW), jnp.float32),
            pltpu.VMEM((CH, 16), jnp.float32),
            pltpu.VMEM((CH,), jnp.int32),
            pltpu.VMEM((CH,), jnp.int32),
            pltpu.VMEM((CH, W2ROW), jnp.float32),
            pltpu.VMEM((CH, 16), jnp.float32),
            pltpu.VMEM_SHARED((N_ACC, W2ROW), jnp.float32),
            pltpu.SemaphoreType.DMA,
            pltpu.SemaphoreType.DMA,
            pltpu.SemaphoreType.DMA,
            pltpu.SemaphoreType.DMA,
        ],
    )(_sc2_body)
    return fn(gtab, ltab2, srcs, dsts, zrows)


def _tc3_body(top_ref, bot_ref, out_ref):
    s = top_ref[...] + bot_ref[...]
    hp = s[:, :NCLASS] / (s[:, NCLASS:NCLASS + 1] + 1e-16)
    o = jnp.where(hp > 0, hp, jnp.exp(hp) - 1.0)
    m = jnp.max(o, axis=1, keepdims=True)
    lse = jnp.log(jnp.sum(jnp.exp(o - m), axis=1, keepdims=True)) + m
    out_ref[...] = o - lse


def _tc3(acc2):
    return pl.pallas_call(
        _tc3_body,
        grid=(NS,),
        in_specs=[
            pl.BlockSpec((RPT, W2ROW), lambda i: (i, 0)),
            pl.BlockSpec((RPT, W2ROW), lambda i: (NS + i, 0)),
        ],
        out_specs=pl.BlockSpec((RPT, NCLASS), lambda i: (i, 0)),
        out_shape=jax.ShapeDtypeStruct((N_ACC, NCLASS), jnp.float32),
    )(acc2, acc2)


def kernel(x, adj, W0, a0, W1, a1, W2, a2, W3, a3, W_out, a_out):
    xp = jnp.pad(x, ((0, N_PAD - N_NODES), (0, 0)))
    src = adj[0]
    dst = adj[1]

    Wc = jnp.stack([jnp.concatenate([W0, W1], axis=1),
                    jnp.concatenate([W2, W3], axis=1)])
    als = [a[0, :NHID] for a in (a0, a1, a2, a3)]
    ars = [a[0, NHID:] for a in (a0, a1, a2, a3)]
    LRm = jnp.zeros((NC, HPW, 16), jnp.float32)
    for c in range(NC):
        LRm = LRm.at[c, :NHID, 0].set(als[2 * c])
        LRm = LRm.at[c, NHID:, 1].set(als[2 * c + 1])
        LRm = LRm.at[c, :NHID, 2].set(ars[2 * c])
        LRm = LRm.at[c, NHID:, 3].set(ars[2 * c + 1])
    AolM = jnp.zeros((NCLASS, 16), jnp.float32).at[:, 0].set(a_out[0, :NCLASS])
    AorM = jnp.zeros((NCLASS, 16), jnp.float32).at[:, 0].set(a_out[0, NCLASS:])

    zd = jnp.zeros((RPT, HPW), jnp.float32)
    zw = jnp.zeros((RPT, 16), jnp.float32)
    zr2 = jnp.zeros((RPT, W2ROW), jnp.float32)

    htab, lrtab = _tc1(xp, Wc, LRm)
    accd, accw = _sc1(htab, lrtab, src, dst, zd, zw)
    gtab, ltab2 = _tc2(accd, accw, W_out, AorM, AolM)
    acc2 = _sc2(gtab, ltab2, src, dst, zr2)
    outp = _tc3(acc2)
    return outp[:N_NODES]
